# baseline (speedup 1.0000x reference)
import functools

import jax
import jax.numpy as jnp
from jax import lax
from jax.experimental import pallas as pl
from jax.experimental.pallas import tpu as pltpu
from jax.experimental.pallas import tpu_sc as plsc

K = 20
B = 4
N = 1024
BN = B * N
CNT = BN * K
CP = 128
PB = 512
NPB = BN // PB

_f32 = jnp.float32
_bf16 = jnp.bfloat16


def _lrelu(z):
    return jnp.where(z > 0, z, 0.2 * z)


def _epilogue(S1, S2, MX, MN, g, b):
    m = jnp.sum(S1, axis=0, keepdims=True) * (1.0 / CNT)
    v = jnp.sum(S2, axis=0, keepdims=True) * (1.0 / CNT) - m * m
    inv = g / jnp.sqrt(v + 1e-5)
    sel = jnp.where(g >= 0, MX, MN)
    z = (sel - m) * inv + b
    return _lrelu(z)


def _topk_writeidx(x_all, idx_ref, pd_ref):
    iota = lax.broadcasted_iota(jnp.int32, (N, N), 1)
    iok = lax.broadcasted_iota(jnp.int32, (N, K), 1)
    for bb in range(B):
        xb = lax.slice(x_all, (bb * N, 0), ((bb + 1) * N, x_all.shape[1]))
        xx = jnp.sum(xb * xb, axis=1, keepdims=True)
        xb16 = xb.astype(_bf16)
        dot = lax.dot_general(xb16, xb16, (((1,), (1,)), ((), ())),
                              preferred_element_type=_f32)
        pd_ref[...] = 2.0 * dot - xx - jnp.transpose(xx)

        def body(j, carry):
            am_prev, acc = carry
            nv = jnp.where(iota == am_prev, -jnp.inf, pd_ref[...])
            pd_ref[...] = nv
            rmax = jnp.max(nv, axis=1, keepdims=True)
            cand = jnp.where(nv == rmax, iota, N)
            am = jnp.min(cand, axis=1, keepdims=True)
            return am, jnp.where(iok == j, am, acc)

        _, idxb = lax.fori_loop(
            0, K, body,
            (jnp.full((N, 1), -1, jnp.int32), jnp.zeros((N, K), jnp.int32)))
        idx_ref[pl.ds(bb * N, N), :] = idxb + bb * N


def _first_body(x_ref, idx_ref, pd_ref):
    _topk_writeidx(x_ref[...], idx_ref, pd_ref)


def _mid_body(s1_ref, s2_ref, mx_ref, mn_ref, g_ref, b_ref,
              x_ref, idx_ref, pd_ref):
    xv = _epilogue(s1_ref[...], s2_ref[...], mx_ref[...], mn_ref[...],
                   g_ref[...], b_ref[...])
    x_ref[...] = xv
    _topk_writeidx(xv, idx_ref, pd_ref)


def _epi_body(s1_ref, s2_ref, mx_ref, mn_ref, g_ref, b_ref, x_ref):
    x_ref[...] = _epilogue(s1_ref[...], s2_ref[...], mx_ref[...], mn_ref[...],
                           g_ref[...], b_ref[...])


def _edge_body(gx_ref, x_ref, w_ref, s1_ref, s2_ref, mx_ref, mn_ref):
    k = pl.program_id(1)
    xv = x_ref[...]
    f = jnp.concatenate([gx_ref[...] - xv, xv], axis=1).astype(_bf16)
    y = jnp.dot(f, w_ref[...], preferred_element_type=_f32)
    y2 = y * y

    @pl.when(k == 0)
    def _():
        s1_ref[...] = y
        s2_ref[...] = y2
        mx_ref[...] = y
        mn_ref[...] = y

    @pl.when(k != 0)
    def _():
        s1_ref[...] += y
        s2_ref[...] += y2
        mx_ref[...] = jnp.maximum(mx_ref[...], y)
        mn_ref[...] = jnp.minimum(mn_ref[...], y)


def _final_body(x1_ref, x2_ref, x3_ref, x4_ref,
                w5a_ref, w5b_ref, w5c_ref, w5d_ref,
                g5_ref, b5_ref, gvec_ref, y_ref):
    for bb in range(B):
        sl = pl.ds(bb * N, N)
        yb = (jnp.dot(x1_ref[sl, :].astype(_bf16), w5a_ref[...],
                      preferred_element_type=_f32)
              + jnp.dot(x2_ref[sl, :].astype(_bf16), w5b_ref[...],
                        preferred_element_type=_f32)
              + jnp.dot(x3_ref[sl, :].astype(_bf16), w5c_ref[...],
                        preferred_element_type=_f32)
              + jnp.dot(x4_ref[sl, :].astype(_bf16), w5d_ref[...],
                        preferred_element_type=_f32))
        y_ref[sl, :] = yb
    ysum = jnp.zeros((1, 1024), _f32)
    for bb in range(B):
        ysum = ysum + jnp.sum(y_ref[pl.ds(bb * N, N), :], axis=0, keepdims=True)
    m = ysum * (1.0 / BN)
    vsum = jnp.zeros((1, 1024), _f32)
    for bb in range(B):
        d = y_ref[pl.ds(bb * N, N), :] - m
        vsum = vsum + jnp.sum(d * d, axis=0, keepdims=True)
    v = vsum * (1.0 / BN)
    g5 = g5_ref[...]
    b5 = b5_ref[...]
    inv = g5 / jnp.sqrt(v + 1e-5)
    for bb in range(B):
        blk = y_ref[pl.ds(bb * N, N), :]
        sel = jnp.where(g5 >= 0,
                        jnp.max(blk, axis=0, keepdims=True),
                        jnp.min(blk, axis=0, keepdims=True))
        p1 = _lrelu((sel - m) * inv + b5)
        act = _lrelu((blk - m) * inv + b5)
        p2 = jnp.sum(act, axis=0, keepdims=True) * (1.0 / N)
        gvec_ref[pl.ds(bb, 1), pl.ds(0, 1024)] = p1
        gvec_ref[pl.ds(bb, 1), pl.ds(1024, 1024)] = p2


def _tc_first(x0p):
    return pl.pallas_call(
        _first_body,
        out_shape=jax.ShapeDtypeStruct((BN, K), jnp.int32),
        scratch_shapes=[pltpu.VMEM((N, N), _f32)],
    )(x0p)


def _tc_mid(S1, S2, MX, MN, g, b):
    return pl.pallas_call(
        _mid_body,
        out_shape=(
            jax.ShapeDtypeStruct((BN, CP), _f32),
            jax.ShapeDtypeStruct((BN, K), jnp.int32),
        ),
        scratch_shapes=[pltpu.VMEM((N, N), _f32)],
    )(S1, S2, MX, MN, g, b)


def _tc_epi(S1, S2, MX, MN, g, b, opad):
    return pl.pallas_call(
        _epi_body,
        out_shape=jax.ShapeDtypeStruct((BN, opad), _f32),
    )(S1, S2, MX, MN, g, b)


def _tc_edge(gx, x, w16, opad):
    return pl.pallas_call(
        _edge_body,
        grid=(NPB, K),
        in_specs=[
            pl.BlockSpec((PB, CP), lambda i, k: (k * NPB + i, 0)),
            pl.BlockSpec((PB, CP), lambda i, k: (i, 0)),
            pl.BlockSpec((2 * CP, opad), lambda i, k: (0, 0)),
        ],
        out_specs=[
            pl.BlockSpec((PB, opad), lambda i, k: (i, 0)),
            pl.BlockSpec((PB, opad), lambda i, k: (i, 0)),
            pl.BlockSpec((PB, opad), lambda i, k: (i, 0)),
            pl.BlockSpec((PB, opad), lambda i, k: (i, 0)),
        ],
        out_shape=(
            jax.ShapeDtypeStruct((BN, opad), _f32),
            jax.ShapeDtypeStruct((BN, opad), _f32),
            jax.ShapeDtypeStruct((BN, opad), _f32),
            jax.ShapeDtypeStruct((BN, opad), _f32),
        ),
    )(gx, x, w16)


def _tc_final(x1, x2, x3, x4, w5a, w5b, w5c, w5d, g5, b5):
    return pl.pallas_call(
        _final_body,
        out_shape=jax.ShapeDtypeStruct((B, 2048), _f32),
        scratch_shapes=[pltpu.VMEM((BN, 1024), _f32)],
    )(x1, x2, x3, x4, w5a, w5b, w5c, w5d, g5, b5)


_NW = 32
_RW = BN * K // _NW
_GR = 128
_NG = _RW // _GR


def _sc_gather_body(x_hbm, idx_hbm, gx_hbm, idx_v, rows_v, sem):
    wid = lax.axis_index("s") * 2 + lax.axis_index("c")
    base = wid * _RW
    pltpu.sync_copy(idx_hbm.at[pl.ds(base, _RW)], idx_v)

    def g_body(g2, _):
        off = g2 * _GR
        pltpu.async_copy(x_hbm.at[idx_v.at[pl.ds(off, _GR)]], rows_v, sem).wait()
        pltpu.sync_copy(rows_v, gx_hbm.at[pl.ds(base + off, _GR)])
        return 0

    lax.fori_loop(0, _NG, g_body, 0)


@functools.lru_cache(maxsize=None)
def _make_sc_gather():
    mesh = plsc.VectorSubcoreMesh(core_axis_name="c", subcore_axis_name="s")
    return functools.partial(
        pl.kernel,
        out_type=jax.ShapeDtypeStruct((BN * K, CP), _f32),
        mesh=mesh,
        scratch_types=[
            pltpu.VMEM((_RW,), jnp.int32),
            pltpu.VMEM((_GR, CP), _f32),
            pltpu.SemaphoreType.DMA,
        ],
    )(_sc_gather_body)


def _sc_gather(x, idx_km):
    return _make_sc_gather()(x, idx_km)


def _prep_w16(W, opad):
    O, twoC = W.shape
    C = twoC // 2
    w = jnp.zeros((2 * CP, opad), _f32)
    w = w.at[:C, :O].set(W[:, :C].T)
    w = w.at[CP:CP + C, :O].set(W[:, C:].T)
    return w.astype(_bf16)


def _pad_vec(g, opad):
    return jnp.zeros((1, opad), _f32).at[0, : g.shape[0]].set(g)


def _kmajor(idx):
    return jnp.transpose(idx).reshape(-1)


def kernel(xyz, W1, g1, b1, W2, g2, b2, W3, g3, b3, W4, g4, b4, W5, g5, b5):
    x0p = jnp.pad(xyz.reshape(BN, 3), ((0, 0), (0, CP - 3)))

    idx1 = _tc_first(x0p)
    gx = _sc_gather(x0p, _kmajor(idx1))
    S1, S2, MX, MN = _tc_edge(gx, x0p, _prep_w16(W1, 128), 128)

    x1, idx2 = _tc_mid(S1, S2, MX, MN, _pad_vec(g1, 128), _pad_vec(b1, 128))
    gx = _sc_gather(x1, _kmajor(idx2))
    S1, S2, MX, MN = _tc_edge(gx, x1, _prep_w16(W2, 128), 128)

    x2, idx3 = _tc_mid(S1, S2, MX, MN, _pad_vec(g2, 128), _pad_vec(b2, 128))
    gx = _sc_gather(x2, _kmajor(idx3))
    S1, S2, MX, MN = _tc_edge(gx, x2, _prep_w16(W3, 128), 128)

    x3, idx4 = _tc_mid(S1, S2, MX, MN, _pad_vec(g3, 128), _pad_vec(b3, 128))
    gx = _sc_gather(x3, _kmajor(idx4))
    S1, S2, MX, MN = _tc_edge(gx, x3, _prep_w16(W4, 384), 384)

    x4 = _tc_epi(S1, S2, MX, MN, _pad_vec(g4, 384), _pad_vec(b4, 384), 384)

    w5a = jnp.zeros((128, 1024), _f32).at[:64, :].set(W5[:, :64].T).astype(_bf16)
    w5b = jnp.zeros((128, 1024), _f32).at[:64, :].set(W5[:, 64:128].T).astype(_bf16)
    w5c = jnp.asarray(W5[:, 128:256].T, _f32).astype(_bf16)
    w5d = jnp.zeros((384, 1024), _f32).at[:257, :].set(W5[:, 256:513].T).astype(_bf16)

    gvec = _tc_final(x1, x2, x3, x4, w5a, w5b, w5c, w5d,
                     _pad_vec(g5, 1024), _pad_vec(b5, 1024))

    xf = jnp.concatenate([x1[:, :64], x2[:, :64], x3, x4[:, :257]], axis=1)
    xft = jnp.transpose(xf.reshape(B, N, 513), (0, 2, 1))
    x0t = jnp.transpose(xyz, (0, 2, 1))
    gr = jnp.broadcast_to(gvec[:, :, None], (B, 2048, N))
    return jnp.concatenate([gr, x0t, xft], axis=1)


# speedup vs baseline: 1.0160x; 1.0160x over previous
import functools

import jax
import jax.numpy as jnp
from jax import lax
from jax.experimental import pallas as pl
from jax.experimental.pallas import tpu as pltpu
from jax.experimental.pallas import tpu_sc as plsc

K = 20
B = 4
N = 1024
BN = B * N
CNT = BN * K
CP = 128
PB = 512
NPB = BN // PB

_f32 = jnp.float32
_bf16 = jnp.bfloat16


def _lrelu(z):
    return jnp.where(z > 0, z, 0.2 * z)


def _epilogue(S1, S2, MX, MN, g, b):
    m = jnp.sum(S1, axis=0, keepdims=True) * (1.0 / CNT)
    v = jnp.sum(S2, axis=0, keepdims=True) * (1.0 / CNT) - m * m
    inv = g / jnp.sqrt(v + 1e-5)
    sel = jnp.where(g >= 0, MX, MN)
    z = (sel - m) * inv + b
    return _lrelu(z)


def _topk_writeidx(x_all, idx_ref, pd_ref):
    iota = lax.broadcasted_iota(jnp.int32, (N, N), 1)
    iok = lax.broadcasted_iota(jnp.int32, (N, K), 1)
    for bb in range(B):
        xb = lax.slice(x_all, (bb * N, 0), ((bb + 1) * N, x_all.shape[1]))
        xx = jnp.sum(xb * xb, axis=1, keepdims=True)
        xb16 = xb.astype(_bf16)
        dot = lax.dot_general(xb16, xb16, (((1,), (1,)), ((), ())),
                              preferred_element_type=_f32)
        pd_ref[...] = 2.0 * dot - xx - jnp.transpose(xx)

        def body(j, acc):
            nv = pd_ref[...]
            rmax = jnp.max(nv, axis=1, keepdims=True)
            cand = jnp.where(nv == rmax, iota, N)
            am = jnp.min(cand, axis=1, keepdims=True)
            pd_ref[...] = jnp.where(iota == am, -jnp.inf, nv)
            return jnp.where(iok == j, am, acc)

        idxb = lax.fori_loop(0, K, body, jnp.zeros((N, K), jnp.int32))
        idx_ref[pl.ds(bb * N, N), :] = idxb + bb * N


def _first_body(x_ref, idx_ref, pd_ref):
    _topk_writeidx(x_ref[...], idx_ref, pd_ref)


def _mid_body(s1_ref, s2_ref, mx_ref, mn_ref, g_ref, b_ref,
              x_ref, idx_ref, pd_ref):
    xv = _epilogue(s1_ref[...], s2_ref[...], mx_ref[...], mn_ref[...],
                   g_ref[...], b_ref[...])
    x_ref[...] = xv
    _topk_writeidx(xv, idx_ref, pd_ref)


def _epi_body(s1_ref, s2_ref, mx_ref, mn_ref, g_ref, b_ref, x_ref):
    x_ref[...] = _epilogue(s1_ref[...], s2_ref[...], mx_ref[...], mn_ref[...],
                           g_ref[...], b_ref[...])


def _edge_body(gx_ref, x_ref, w_ref, s1_ref, s2_ref, mx_ref, mn_ref):
    k = pl.program_id(1)
    xv = x_ref[...]
    f = jnp.concatenate([gx_ref[...] - xv, xv], axis=1).astype(_bf16)
    y = jnp.dot(f, w_ref[...], preferred_element_type=_f32)
    y2 = y * y

    @pl.when(k == 0)
    def _():
        s1_ref[...] = y
        s2_ref[...] = y2
        mx_ref[...] = y
        mn_ref[...] = y

    @pl.when(k != 0)
    def _():
        s1_ref[...] += y
        s2_ref[...] += y2
        mx_ref[...] = jnp.maximum(mx_ref[...], y)
        mn_ref[...] = jnp.minimum(mn_ref[...], y)


def _final_body(x1_ref, x2_ref, x3_ref, x4_ref,
                w5a_ref, w5b_ref, w5c_ref, w5d_ref,
                g5_ref, b5_ref, gvec_ref, y_ref):
    for bb in range(B):
        sl = pl.ds(bb * N, N)
        yb = (jnp.dot(x1_ref[sl, :].astype(_bf16), w5a_ref[...],
                      preferred_element_type=_f32)
              + jnp.dot(x2_ref[sl, :].astype(_bf16), w5b_ref[...],
                        preferred_element_type=_f32)
              + jnp.dot(x3_ref[sl, :].astype(_bf16), w5c_ref[...],
                        preferred_element_type=_f32)
              + jnp.dot(x4_ref[sl, :].astype(_bf16), w5d_ref[...],
                        preferred_element_type=_f32))
        y_ref[sl, :] = yb
    ysum = jnp.zeros((1, 1024), _f32)
    for bb in range(B):
        ysum = ysum + jnp.sum(y_ref[pl.ds(bb * N, N), :], axis=0, keepdims=True)
    m = ysum * (1.0 / BN)
    vsum = jnp.zeros((1, 1024), _f32)
    for bb in range(B):
        d = y_ref[pl.ds(bb * N, N), :] - m
        vsum = vsum + jnp.sum(d * d, axis=0, keepdims=True)
    v = vsum * (1.0 / BN)
    g5 = g5_ref[...]
    b5 = b5_ref[...]
    inv = g5 / jnp.sqrt(v + 1e-5)
    for bb in range(B):
        blk = y_ref[pl.ds(bb * N, N), :]
        sel = jnp.where(g5 >= 0,
                        jnp.max(blk, axis=0, keepdims=True),
                        jnp.min(blk, axis=0, keepdims=True))
        p1 = _lrelu((sel - m) * inv + b5)
        act = _lrelu((blk - m) * inv + b5)
        p2 = jnp.sum(act, axis=0, keepdims=True) * (1.0 / N)
        gvec_ref[pl.ds(bb, 1), pl.ds(0, 1024)] = p1
        gvec_ref[pl.ds(bb, 1), pl.ds(1024, 1024)] = p2


def _tc_first(x0p):
    return pl.pallas_call(
        _first_body,
        out_shape=jax.ShapeDtypeStruct((BN, K), jnp.int32),
        scratch_shapes=[pltpu.VMEM((N, N), _f32)],
    )(x0p)


def _tc_mid(S1, S2, MX, MN, g, b):
    return pl.pallas_call(
        _mid_body,
        out_shape=(
            jax.ShapeDtypeStruct((BN, CP), _f32),
            jax.ShapeDtypeStruct((BN, K), jnp.int32),
        ),
        scratch_shapes=[pltpu.VMEM((N, N), _f32)],
    )(S1, S2, MX, MN, g, b)


def _tc_epi(S1, S2, MX, MN, g, b, opad):
    return pl.pallas_call(
        _epi_body,
        out_shape=jax.ShapeDtypeStruct((BN, opad), _f32),
    )(S1, S2, MX, MN, g, b)


def _tc_edge(gx, x, w16, opad):
    return pl.pallas_call(
        _edge_body,
        grid=(NPB, K),
        in_specs=[
            pl.BlockSpec((PB, CP), lambda i, k: (k * NPB + i, 0)),
            pl.BlockSpec((PB, CP), lambda i, k: (i, 0)),
            pl.BlockSpec((2 * CP, opad), lambda i, k: (0, 0)),
        ],
        out_specs=[
            pl.BlockSpec((PB, opad), lambda i, k: (i, 0)),
            pl.BlockSpec((PB, opad), lambda i, k: (i, 0)),
            pl.BlockSpec((PB, opad), lambda i, k: (i, 0)),
            pl.BlockSpec((PB, opad), lambda i, k: (i, 0)),
        ],
        out_shape=(
            jax.ShapeDtypeStruct((BN, opad), _f32),
            jax.ShapeDtypeStruct((BN, opad), _f32),
            jax.ShapeDtypeStruct((BN, opad), _f32),
            jax.ShapeDtypeStruct((BN, opad), _f32),
        ),
    )(gx, x, w16)


def _tc_final(x1, x2, x3, x4, w5a, w5b, w5c, w5d, g5, b5):
    return pl.pallas_call(
        _final_body,
        out_shape=jax.ShapeDtypeStruct((B, 2048), _f32),
        scratch_shapes=[pltpu.VMEM((BN, 1024), _f32)],
    )(x1, x2, x3, x4, w5a, w5b, w5c, w5d, g5, b5)


_NW = 32
_RW = BN * K // _NW
_GR = 128
_NG = _RW // _GR


def _sc_gather_body(x_hbm, idx_hbm, gx_hbm, idx_v, rows_v, sem):
    wid = lax.axis_index("s") * 2 + lax.axis_index("c")
    base = wid * _RW
    pltpu.sync_copy(idx_hbm.at[pl.ds(base, _RW)], idx_v)

    def g_body(g2, _):
        off = g2 * _GR
        pltpu.async_copy(x_hbm.at[idx_v.at[pl.ds(off, _GR)]], rows_v, sem).wait()
        pltpu.sync_copy(rows_v, gx_hbm.at[pl.ds(base + off, _GR)])
        return 0

    lax.fori_loop(0, _NG, g_body, 0)


@functools.lru_cache(maxsize=None)
def _make_sc_gather():
    mesh = plsc.VectorSubcoreMesh(core_axis_name="c", subcore_axis_name="s")
    return functools.partial(
        pl.kernel,
        out_type=jax.ShapeDtypeStruct((BN * K, CP), _f32),
        mesh=mesh,
        scratch_types=[
            pltpu.VMEM((_RW,), jnp.int32),
            pltpu.VMEM((_GR, CP), _f32),
            pltpu.SemaphoreType.DMA,
        ],
    )(_sc_gather_body)


def _sc_gather(x, idx_km):
    return _make_sc_gather()(x, idx_km)


def _prep_w16(W, opad):
    O, twoC = W.shape
    C = twoC // 2
    w = jnp.zeros((2 * CP, opad), _f32)
    w = w.at[:C, :O].set(W[:, :C].T)
    w = w.at[CP:CP + C, :O].set(W[:, C:].T)
    return w.astype(_bf16)


def _pad_vec(g, opad):
    return jnp.zeros((1, opad), _f32).at[0, : g.shape[0]].set(g)


def _kmajor(idx):
    return jnp.transpose(idx).reshape(-1)


def kernel(xyz, W1, g1, b1, W2, g2, b2, W3, g3, b3, W4, g4, b4, W5, g5, b5):
    x0p = jnp.pad(xyz.reshape(BN, 3), ((0, 0), (0, CP - 3)))

    idx1 = _tc_first(x0p)
    gx = _sc_gather(x0p, _kmajor(idx1))
    S1, S2, MX, MN = _tc_edge(gx, x0p, _prep_w16(W1, 128), 128)

    x1, idx2 = _tc_mid(S1, S2, MX, MN, _pad_vec(g1, 128), _pad_vec(b1, 128))
    gx = _sc_gather(x1, _kmajor(idx2))
    S1, S2, MX, MN = _tc_edge(gx, x1, _prep_w16(W2, 128), 128)

    x2, idx3 = _tc_mid(S1, S2, MX, MN, _pad_vec(g2, 128), _pad_vec(b2, 128))
    gx = _sc_gather(x2, _kmajor(idx3))
    S1, S2, MX, MN = _tc_edge(gx, x2, _prep_w16(W3, 128), 128)

    x3, idx4 = _tc_mid(S1, S2, MX, MN, _pad_vec(g3, 128), _pad_vec(b3, 128))
    gx = _sc_gather(x3, _kmajor(idx4))
    S1, S2, MX, MN = _tc_edge(gx, x3, _prep_w16(W4, 384), 384)

    x4 = _tc_epi(S1, S2, MX, MN, _pad_vec(g4, 384), _pad_vec(b4, 384), 384)

    w5a = jnp.zeros((128, 1024), _f32).at[:64, :].set(W5[:, :64].T).astype(_bf16)
    w5b = jnp.zeros((128, 1024), _f32).at[:64, :].set(W5[:, 64:128].T).astype(_bf16)
    w5c = jnp.asarray(W5[:, 128:256].T, _f32).astype(_bf16)
    w5d = jnp.zeros((384, 1024), _f32).at[:257, :].set(W5[:, 256:513].T).astype(_bf16)

    gvec = _tc_final(x1, x2, x3, x4, w5a, w5b, w5c, w5d,
                     _pad_vec(g5, 1024), _pad_vec(b5, 1024))

    xf = jnp.concatenate([x1[:, :64], x2[:, :64], x3, x4[:, :257]], axis=1)
    xft = jnp.transpose(xf.reshape(B, N, 513), (0, 2, 1))
    x0t = jnp.transpose(xyz, (0, 2, 1))
    gr = jnp.broadcast_to(gvec[:, :, None], (B, 2048, N))
    return jnp.concatenate([gr, x0t, xft], axis=1)


# speedup vs baseline: 1.2388x; 1.2193x over previous
import functools

import jax
import jax.numpy as jnp
from jax import lax
from jax.experimental import pallas as pl
from jax.experimental.pallas import tpu as pltpu
from jax.experimental.pallas import tpu_sc as plsc

K = 20
B = 4
N = 1024
BN = B * N
CNT = BN * K
CP = 128
PB = 2048
NPB = BN // PB

_f32 = jnp.float32
_bf16 = jnp.bfloat16


def _lrelu(z):
    return jnp.where(z > 0, z, 0.2 * z)


def _epilogue(S1, S2, MX, MN, g, b):
    m = jnp.sum(S1, axis=0, keepdims=True) * (1.0 / CNT)
    v = jnp.sum(S2, axis=0, keepdims=True) * (1.0 / CNT) - m * m
    inv = g / jnp.sqrt(v + 1e-5)
    sel = jnp.where(g >= 0, MX, MN)
    z = (sel - m) * inv + b
    return _lrelu(z)


def _topk_writeidx(x_all, idx_ref, pd_ref):
    iota = lax.broadcasted_iota(jnp.int32, (N, N), 1)
    iok = lax.broadcasted_iota(jnp.int32, (N, K), 1)
    for bb in range(B):
        xb = lax.slice(x_all, (bb * N, 0), ((bb + 1) * N, x_all.shape[1]))
        xx = jnp.sum(xb * xb, axis=1, keepdims=True)
        xb16 = xb.astype(_bf16)
        dot = lax.dot_general(xb16, xb16, (((1,), (1,)), ((), ())),
                              preferred_element_type=_f32)
        pd_ref[...] = 2.0 * dot - xx - jnp.transpose(xx)

        def body(j, acc):
            nv = pd_ref[...]
            rmax = jnp.max(nv, axis=1, keepdims=True)
            cand = jnp.where(nv == rmax, iota, N)
            am = jnp.min(cand, axis=1, keepdims=True)
            pd_ref[...] = jnp.where(iota == am, -jnp.inf, nv)
            return jnp.where(iok == j, am, acc)

        idxb = lax.fori_loop(0, K, body, jnp.zeros((N, K), jnp.int32))
        idx_ref[pl.ds(bb * N, N), :] = idxb + bb * N


def _first_body(x_ref, idx_ref, pd_ref):
    _topk_writeidx(x_ref[...], idx_ref, pd_ref)


def _mid_body(s1_ref, s2_ref, mx_ref, mn_ref, g_ref, b_ref,
              x_ref, idx_ref, pd_ref):
    xv = _epilogue(s1_ref[...], s2_ref[...], mx_ref[...], mn_ref[...],
                   g_ref[...], b_ref[...])
    x_ref[...] = xv
    _topk_writeidx(xv, idx_ref, pd_ref)


def _epi_body(s1_ref, s2_ref, mx_ref, mn_ref, g_ref, b_ref, x_ref):
    x_ref[...] = _epilogue(s1_ref[...], s2_ref[...], mx_ref[...], mn_ref[...],
                           g_ref[...], b_ref[...])


def _edge_body(gx_ref, x_ref, w_ref, s1_ref, s2_ref, mx_ref, mn_ref):
    k = pl.program_id(1)
    xv = x_ref[...]
    f = jnp.concatenate([gx_ref[...] - xv, xv], axis=1).astype(_bf16)
    y = jnp.dot(f, w_ref[...], preferred_element_type=_f32)
    y2 = y * y

    @pl.when(k == 0)
    def _():
        s1_ref[...] = y
        s2_ref[...] = y2
        mx_ref[...] = y
        mn_ref[...] = y

    @pl.when(k != 0)
    def _():
        s1_ref[...] += y
        s2_ref[...] += y2
        mx_ref[...] = jnp.maximum(mx_ref[...], y)
        mn_ref[...] = jnp.minimum(mn_ref[...], y)


def _final_body(x1_ref, x2_ref, x3_ref, x4_ref,
                w5a_ref, w5b_ref, w5c_ref, w5d_ref,
                g5_ref, b5_ref, gvec_ref, y_ref):
    for bb in range(B):
        sl = pl.ds(bb * N, N)
        yb = (jnp.dot(x1_ref[sl, :].astype(_bf16), w5a_ref[...],
                      preferred_element_type=_f32)
              + jnp.dot(x2_ref[sl, :].astype(_bf16), w5b_ref[...],
                        preferred_element_type=_f32)
              + jnp.dot(x3_ref[sl, :].astype(_bf16), w5c_ref[...],
                        preferred_element_type=_f32)
              + jnp.dot(x4_ref[sl, :].astype(_bf16), w5d_ref[...],
                        preferred_element_type=_f32))
        y_ref[sl, :] = yb
    ysum = jnp.zeros((1, 1024), _f32)
    for bb in range(B):
        ysum = ysum + jnp.sum(y_ref[pl.ds(bb * N, N), :], axis=0, keepdims=True)
    m = ysum * (1.0 / BN)
    vsum = jnp.zeros((1, 1024), _f32)
    for bb in range(B):
        d = y_ref[pl.ds(bb * N, N), :] - m
        vsum = vsum + jnp.sum(d * d, axis=0, keepdims=True)
    v = vsum * (1.0 / BN)
    g5 = g5_ref[...]
    b5 = b5_ref[...]
    inv = g5 / jnp.sqrt(v + 1e-5)
    for bb in range(B):
        blk = y_ref[pl.ds(bb * N, N), :]
        sel = jnp.where(g5 >= 0,
                        jnp.max(blk, axis=0, keepdims=True),
                        jnp.min(blk, axis=0, keepdims=True))
        p1 = _lrelu((sel - m) * inv + b5)
        act = _lrelu((blk - m) * inv + b5)
        p2 = jnp.sum(act, axis=0, keepdims=True) * (1.0 / N)
        gvec_ref[pl.ds(bb, 1), pl.ds(0, 1024)] = p1
        gvec_ref[pl.ds(bb, 1), pl.ds(1024, 1024)] = p2


def _tc_first(x0p):
    return pl.pallas_call(
        _first_body,
        out_shape=jax.ShapeDtypeStruct((BN, K), jnp.int32),
        scratch_shapes=[pltpu.VMEM((N, N), _f32)],
    )(x0p)


def _tc_mid(S1, S2, MX, MN, g, b):
    return pl.pallas_call(
        _mid_body,
        out_shape=(
            jax.ShapeDtypeStruct((BN, CP), _f32),
            jax.ShapeDtypeStruct((BN, K), jnp.int32),
        ),
        scratch_shapes=[pltpu.VMEM((N, N), _f32)],
    )(S1, S2, MX, MN, g, b)


def _tc_epi(S1, S2, MX, MN, g, b, opad):
    return pl.pallas_call(
        _epi_body,
        out_shape=jax.ShapeDtypeStruct((BN, opad), _f32),
    )(S1, S2, MX, MN, g, b)


def _tc_edge(gx, x, w16, opad):
    return pl.pallas_call(
        _edge_body,
        grid=(NPB, K),
        in_specs=[
            pl.BlockSpec((PB, CP), lambda i, k: (k * NPB + i, 0)),
            pl.BlockSpec((PB, CP), lambda i, k: (i, 0)),
            pl.BlockSpec((2 * CP, opad), lambda i, k: (0, 0)),
        ],
        out_specs=[
            pl.BlockSpec((PB, opad), lambda i, k: (i, 0)),
            pl.BlockSpec((PB, opad), lambda i, k: (i, 0)),
            pl.BlockSpec((PB, opad), lambda i, k: (i, 0)),
            pl.BlockSpec((PB, opad), lambda i, k: (i, 0)),
        ],
        out_shape=(
            jax.ShapeDtypeStruct((BN, opad), _f32),
            jax.ShapeDtypeStruct((BN, opad), _f32),
            jax.ShapeDtypeStruct((BN, opad), _f32),
            jax.ShapeDtypeStruct((BN, opad), _f32),
        ),
    )(gx, x, w16)


def _tc_final(x1, x2, x3, x4, w5a, w5b, w5c, w5d, g5, b5):
    return pl.pallas_call(
        _final_body,
        out_shape=jax.ShapeDtypeStruct((B, 2048), _f32),
        scratch_shapes=[pltpu.VMEM((BN, 1024), _f32)],
    )(x1, x2, x3, x4, w5a, w5b, w5c, w5d, g5, b5)


_NW = 32
_RW = BN * K // _NW
_GR = 128
_NG = _RW // _GR


def _sc_gather_body(x_hbm, idx_hbm, gx_hbm, idx_v, rows_v, sem):
    wid = lax.axis_index("s") * 2 + lax.axis_index("c")
    base = wid * _RW
    pltpu.sync_copy(idx_hbm.at[pl.ds(base, _RW)], idx_v)

    def g_body(g2, _):
        off = g2 * _GR
        pltpu.async_copy(x_hbm.at[idx_v.at[pl.ds(off, _GR)]], rows_v, sem).wait()
        pltpu.sync_copy(rows_v, gx_hbm.at[pl.ds(base + off, _GR)])
        return 0

    lax.fori_loop(0, _NG, g_body, 0)


@functools.lru_cache(maxsize=None)
def _make_sc_gather():
    mesh = plsc.VectorSubcoreMesh(core_axis_name="c", subcore_axis_name="s")
    return functools.partial(
        pl.kernel,
        out_type=jax.ShapeDtypeStruct((BN * K, CP), _f32),
        mesh=mesh,
        scratch_types=[
            pltpu.VMEM((_RW,), jnp.int32),
            pltpu.VMEM((_GR, CP), _f32),
            pltpu.SemaphoreType.DMA,
        ],
    )(_sc_gather_body)


def _sc_gather(x, idx_km):
    return _make_sc_gather()(x, idx_km)


def _prep_w16(W, opad):
    O, twoC = W.shape
    C = twoC // 2
    w = jnp.zeros((2 * CP, opad), _f32)
    w = w.at[:C, :O].set(W[:, :C].T)
    w = w.at[CP:CP + C, :O].set(W[:, C:].T)
    return w.astype(_bf16)


def _pad_vec(g, opad):
    return jnp.zeros((1, opad), _f32).at[0, : g.shape[0]].set(g)


def _kmajor(idx):
    return jnp.transpose(idx).reshape(-1)


def kernel(xyz, W1, g1, b1, W2, g2, b2, W3, g3, b3, W4, g4, b4, W5, g5, b5):
    x0p = jnp.pad(xyz.reshape(BN, 3), ((0, 0), (0, CP - 3)))

    idx1 = _tc_first(x0p)
    gx = _sc_gather(x0p, _kmajor(idx1))
    S1, S2, MX, MN = _tc_edge(gx, x0p, _prep_w16(W1, 128), 128)

    x1, idx2 = _tc_mid(S1, S2, MX, MN, _pad_vec(g1, 128), _pad_vec(b1, 128))
    gx = _sc_gather(x1, _kmajor(idx2))
    S1, S2, MX, MN = _tc_edge(gx, x1, _prep_w16(W2, 128), 128)

    x2, idx3 = _tc_mid(S1, S2, MX, MN, _pad_vec(g2, 128), _pad_vec(b2, 128))
    gx = _sc_gather(x2, _kmajor(idx3))
    S1, S2, MX, MN = _tc_edge(gx, x2, _prep_w16(W3, 128), 128)

    x3, idx4 = _tc_mid(S1, S2, MX, MN, _pad_vec(g3, 128), _pad_vec(b3, 128))
    gx = _sc_gather(x3, _kmajor(idx4))
    S1, S2, MX, MN = _tc_edge(gx, x3, _prep_w16(W4, 384), 384)

    x4 = _tc_epi(S1, S2, MX, MN, _pad_vec(g4, 384), _pad_vec(b4, 384), 384)

    w5a = jnp.zeros((128, 1024), _f32).at[:64, :].set(W5[:, :64].T).astype(_bf16)
    w5b = jnp.zeros((128, 1024), _f32).at[:64, :].set(W5[:, 64:128].T).astype(_bf16)
    w5c = jnp.asarray(W5[:, 128:256].T, _f32).astype(_bf16)
    w5d = jnp.zeros((384, 1024), _f32).at[:257, :].set(W5[:, 256:513].T).astype(_bf16)

    gvec = _tc_final(x1, x2, x3, x4, w5a, w5b, w5c, w5d,
                     _pad_vec(g5, 1024), _pad_vec(b5, 1024))

    xf = jnp.concatenate([x1[:, :64], x2[:, :64], x3, x4[:, :257]], axis=1)
    xft = jnp.transpose(xf.reshape(B, N, 513), (0, 2, 1))
    x0t = jnp.transpose(xyz, (0, 2, 1))
    gr = jnp.broadcast_to(gvec[:, :, None], (B, 2048, N))
    return jnp.concatenate([gr, x0t, xft], axis=1)


# speedup vs baseline: 1.2708x; 1.0258x over previous
import functools

import jax
import jax.numpy as jnp
from jax import lax
from jax.experimental import pallas as pl
from jax.experimental.pallas import tpu as pltpu
from jax.experimental.pallas import tpu_sc as plsc

K = 20
B = 4
N = 1024
BN = B * N
CNT = BN * K
CP = 128
PB = 4096
NPB = BN // PB

_f32 = jnp.float32
_bf16 = jnp.bfloat16


def _lrelu(z):
    return jnp.where(z > 0, z, 0.2 * z)


def _epilogue(S1, S2, MX, MN, g, b):
    m = jnp.sum(S1, axis=0, keepdims=True) * (1.0 / CNT)
    v = jnp.sum(S2, axis=0, keepdims=True) * (1.0 / CNT) - m * m
    inv = g / jnp.sqrt(v + 1e-5)
    sel = jnp.where(g >= 0, MX, MN)
    z = (sel - m) * inv + b
    return _lrelu(z)


def _topk_writeidx(x_all, idx_ref, pd_ref):
    iota = lax.broadcasted_iota(jnp.int32, (N, N), 1)
    iok = lax.broadcasted_iota(jnp.int32, (N, K), 1)
    for bb in range(B):
        xb = lax.slice(x_all, (bb * N, 0), ((bb + 1) * N, x_all.shape[1]))
        xx = jnp.sum(xb * xb, axis=1, keepdims=True)
        xb16 = xb.astype(_bf16)
        dot = lax.dot_general(xb16, xb16, (((1,), (1,)), ((), ())),
                              preferred_element_type=_f32)
        pd_ref[...] = 2.0 * dot - xx - jnp.transpose(xx)

        def body(j, acc):
            nv = pd_ref[...]
            rmax = jnp.max(nv, axis=1, keepdims=True)
            cand = jnp.where(nv == rmax, iota, N)
            am = jnp.min(cand, axis=1, keepdims=True)
            pd_ref[...] = jnp.where(iota == am, -jnp.inf, nv)
            return jnp.where(iok == j, am, acc)

        idxb = lax.fori_loop(0, K, body, jnp.zeros((N, K), jnp.int32))
        idx_ref[pl.ds(bb * N, N), :] = idxb + bb * N


def _first_body(x_ref, idx_ref, pd_ref):
    _topk_writeidx(x_ref[...], idx_ref, pd_ref)


def _mid_body(s1_ref, s2_ref, mx_ref, mn_ref, g_ref, b_ref,
              x_ref, idx_ref, pd_ref):
    xv = _epilogue(s1_ref[...], s2_ref[...], mx_ref[...], mn_ref[...],
                   g_ref[...], b_ref[...])
    x_ref[...] = xv
    _topk_writeidx(xv, idx_ref, pd_ref)


def _epi_body(s1_ref, s2_ref, mx_ref, mn_ref, g_ref, b_ref, x_ref):
    x_ref[...] = _epilogue(s1_ref[...], s2_ref[...], mx_ref[...], mn_ref[...],
                           g_ref[...], b_ref[...])


def _edge_body(gx_ref, x_ref, w_ref, s1_ref, s2_ref, mx_ref, mn_ref):
    k = pl.program_id(1)
    xv = x_ref[...]
    f = jnp.concatenate([gx_ref[...] - xv, xv], axis=1).astype(_bf16)
    y = jnp.dot(f, w_ref[...], preferred_element_type=_f32)
    y2 = y * y

    @pl.when(k == 0)
    def _():
        s1_ref[...] = y
        s2_ref[...] = y2
        mx_ref[...] = y
        mn_ref[...] = y

    @pl.when(k != 0)
    def _():
        s1_ref[...] += y
        s2_ref[...] += y2
        mx_ref[...] = jnp.maximum(mx_ref[...], y)
        mn_ref[...] = jnp.minimum(mn_ref[...], y)


def _final_body(x1_ref, x2_ref, x3_ref, x4_ref,
                w5a_ref, w5b_ref, w5c_ref, w5d_ref,
                g5_ref, b5_ref, gvec_ref, y_ref):
    for bb in range(B):
        sl = pl.ds(bb * N, N)
        yb = (jnp.dot(x1_ref[sl, :].astype(_bf16), w5a_ref[...],
                      preferred_element_type=_f32)
              + jnp.dot(x2_ref[sl, :].astype(_bf16), w5b_ref[...],
                        preferred_element_type=_f32)
              + jnp.dot(x3_ref[sl, :].astype(_bf16), w5c_ref[...],
                        preferred_element_type=_f32)
              + jnp.dot(x4_ref[sl, :].astype(_bf16), w5d_ref[...],
                        preferred_element_type=_f32))
        y_ref[sl, :] = yb
    ysum = jnp.zeros((1, 1024), _f32)
    for bb in range(B):
        ysum = ysum + jnp.sum(y_ref[pl.ds(bb * N, N), :], axis=0, keepdims=True)
    m = ysum * (1.0 / BN)
    vsum = jnp.zeros((1, 1024), _f32)
    for bb in range(B):
        d = y_ref[pl.ds(bb * N, N), :] - m
        vsum = vsum + jnp.sum(d * d, axis=0, keepdims=True)
    v = vsum * (1.0 / BN)
    g5 = g5_ref[...]
    b5 = b5_ref[...]
    inv = g5 / jnp.sqrt(v + 1e-5)
    for bb in range(B):
        blk = y_ref[pl.ds(bb * N, N), :]
        sel = jnp.where(g5 >= 0,
                        jnp.max(blk, axis=0, keepdims=True),
                        jnp.min(blk, axis=0, keepdims=True))
        p1 = _lrelu((sel - m) * inv + b5)
        act = _lrelu((blk - m) * inv + b5)
        p2 = jnp.sum(act, axis=0, keepdims=True) * (1.0 / N)
        gvec_ref[pl.ds(bb, 1), pl.ds(0, 1024)] = p1
        gvec_ref[pl.ds(bb, 1), pl.ds(1024, 1024)] = p2


def _tc_first(x0p):
    return pl.pallas_call(
        _first_body,
        out_shape=jax.ShapeDtypeStruct((BN, K), jnp.int32),
        scratch_shapes=[pltpu.VMEM((N, N), _f32)],
    )(x0p)


def _tc_mid(S1, S2, MX, MN, g, b):
    return pl.pallas_call(
        _mid_body,
        out_shape=(
            jax.ShapeDtypeStruct((BN, CP), _f32),
            jax.ShapeDtypeStruct((BN, K), jnp.int32),
        ),
        scratch_shapes=[pltpu.VMEM((N, N), _f32)],
    )(S1, S2, MX, MN, g, b)


def _tc_epi(S1, S2, MX, MN, g, b, opad):
    return pl.pallas_call(
        _epi_body,
        out_shape=jax.ShapeDtypeStruct((BN, opad), _f32),
    )(S1, S2, MX, MN, g, b)


def _tc_edge(gx, x, w16, opad):
    return pl.pallas_call(
        _edge_body,
        grid=(NPB, K),
        in_specs=[
            pl.BlockSpec((PB, CP), lambda i, k: (k * NPB + i, 0)),
            pl.BlockSpec((PB, CP), lambda i, k: (i, 0)),
            pl.BlockSpec((2 * CP, opad), lambda i, k: (0, 0)),
        ],
        out_specs=[
            pl.BlockSpec((PB, opad), lambda i, k: (i, 0)),
            pl.BlockSpec((PB, opad), lambda i, k: (i, 0)),
            pl.BlockSpec((PB, opad), lambda i, k: (i, 0)),
            pl.BlockSpec((PB, opad), lambda i, k: (i, 0)),
        ],
        out_shape=(
            jax.ShapeDtypeStruct((BN, opad), _f32),
            jax.ShapeDtypeStruct((BN, opad), _f32),
            jax.ShapeDtypeStruct((BN, opad), _f32),
            jax.ShapeDtypeStruct((BN, opad), _f32),
        ),
    )(gx, x, w16)


def _tc_final(x1, x2, x3, x4, w5a, w5b, w5c, w5d, g5, b5):
    return pl.pallas_call(
        _final_body,
        out_shape=jax.ShapeDtypeStruct((B, 2048), _f32),
        scratch_shapes=[pltpu.VMEM((BN, 1024), _f32)],
    )(x1, x2, x3, x4, w5a, w5b, w5c, w5d, g5, b5)


_NW = 32
_RW = BN * K // _NW
_GR = 128
_NG = _RW // _GR


def _sc_gather_body(x_hbm, idx_hbm, gx_hbm, idx_v, rows_v, sem):
    wid = lax.axis_index("s") * 2 + lax.axis_index("c")
    base = wid * _RW
    pltpu.sync_copy(idx_hbm.at[pl.ds(base, _RW)], idx_v)

    def g_body(g2, _):
        off = g2 * _GR
        pltpu.async_copy(x_hbm.at[idx_v.at[pl.ds(off, _GR)]], rows_v, sem).wait()
        pltpu.sync_copy(rows_v, gx_hbm.at[pl.ds(base + off, _GR)])
        return 0

    lax.fori_loop(0, _NG, g_body, 0)


@functools.lru_cache(maxsize=None)
def _make_sc_gather():
    mesh = plsc.VectorSubcoreMesh(core_axis_name="c", subcore_axis_name="s")
    return functools.partial(
        pl.kernel,
        out_type=jax.ShapeDtypeStruct((BN * K, CP), _f32),
        mesh=mesh,
        scratch_types=[
            pltpu.VMEM((_RW,), jnp.int32),
            pltpu.VMEM((_GR, CP), _f32),
            pltpu.SemaphoreType.DMA,
        ],
    )(_sc_gather_body)


def _sc_gather(x, idx_km):
    return _make_sc_gather()(x, idx_km)


def _prep_w16(W, opad):
    O, twoC = W.shape
    C = twoC // 2
    w = jnp.zeros((2 * CP, opad), _f32)
    w = w.at[:C, :O].set(W[:, :C].T)
    w = w.at[CP:CP + C, :O].set(W[:, C:].T)
    return w.astype(_bf16)


def _pad_vec(g, opad):
    return jnp.zeros((1, opad), _f32).at[0, : g.shape[0]].set(g)


def _kmajor(idx):
    return jnp.transpose(idx).reshape(-1)


def kernel(xyz, W1, g1, b1, W2, g2, b2, W3, g3, b3, W4, g4, b4, W5, g5, b5):
    x0p = jnp.pad(xyz.reshape(BN, 3), ((0, 0), (0, CP - 3)))

    idx1 = _tc_first(x0p)
    gx = _sc_gather(x0p, _kmajor(idx1))
    S1, S2, MX, MN = _tc_edge(gx, x0p, _prep_w16(W1, 128), 128)

    x1, idx2 = _tc_mid(S1, S2, MX, MN, _pad_vec(g1, 128), _pad_vec(b1, 128))
    gx = _sc_gather(x1, _kmajor(idx2))
    S1, S2, MX, MN = _tc_edge(gx, x1, _prep_w16(W2, 128), 128)

    x2, idx3 = _tc_mid(S1, S2, MX, MN, _pad_vec(g2, 128), _pad_vec(b2, 128))
    gx = _sc_gather(x2, _kmajor(idx3))
    S1, S2, MX, MN = _tc_edge(gx, x2, _prep_w16(W3, 128), 128)

    x3, idx4 = _tc_mid(S1, S2, MX, MN, _pad_vec(g3, 128), _pad_vec(b3, 128))
    gx = _sc_gather(x3, _kmajor(idx4))
    S1, S2, MX, MN = _tc_edge(gx, x3, _prep_w16(W4, 384), 384)

    x4 = _tc_epi(S1, S2, MX, MN, _pad_vec(g4, 384), _pad_vec(b4, 384), 384)

    w5a = jnp.zeros((128, 1024), _f32).at[:64, :].set(W5[:, :64].T).astype(_bf16)
    w5b = jnp.zeros((128, 1024), _f32).at[:64, :].set(W5[:, 64:128].T).astype(_bf16)
    w5c = jnp.asarray(W5[:, 128:256].T, _f32).astype(_bf16)
    w5d = jnp.zeros((384, 1024), _f32).at[:257, :].set(W5[:, 256:513].T).astype(_bf16)

    gvec = _tc_final(x1, x2, x3, x4, w5a, w5b, w5c, w5d,
                     _pad_vec(g5, 1024), _pad_vec(b5, 1024))

    xf = jnp.concatenate([x1[:, :64], x2[:, :64], x3, x4[:, :257]], axis=1)
    xft = jnp.transpose(xf.reshape(B, N, 513), (0, 2, 1))
    x0t = jnp.transpose(xyz, (0, 2, 1))
    gr = jnp.broadcast_to(gvec[:, :, None], (B, 2048, N))
    return jnp.concatenate([gr, x0t, xft], axis=1)


# speedup vs baseline: 1.3200x; 1.0387x over previous
import functools

import jax
import jax.numpy as jnp
from jax import lax
from jax.experimental import pallas as pl
from jax.experimental.pallas import tpu as pltpu
from jax.experimental.pallas import tpu_sc as plsc

K = 20
B = 4
N = 1024
BN = B * N
CNT = BN * K
CP = 128
PB = 4096
NPB = BN // PB

_f32 = jnp.float32
_bf16 = jnp.bfloat16


def _lrelu(z):
    return jnp.where(z > 0, z, 0.2 * z)


def _epilogue(S1, S2, MX, MN, g, b):
    m = jnp.sum(S1, axis=0, keepdims=True) * (1.0 / CNT)
    v = jnp.sum(S2, axis=0, keepdims=True) * (1.0 / CNT) - m * m
    inv = g / jnp.sqrt(v + 1e-5)
    sel = jnp.where(g >= 0, MX, MN)
    z = (sel - m) * inv + b
    return _lrelu(z)


def _topk_writeidx(x_all, idx_ref, pd_ref):
    iota = lax.broadcasted_iota(jnp.int32, (N, N), 1)
    iok = lax.broadcasted_iota(jnp.int32, (N, K), 1)
    for bb in range(B):
        xb = lax.slice(x_all, (bb * N, 0), ((bb + 1) * N, x_all.shape[1]))
        xx = jnp.sum(xb * xb, axis=1, keepdims=True)
        xb16 = xb.astype(_bf16)
        dot = lax.dot_general(xb16, xb16, (((1,), (1,)), ((), ())),
                              preferred_element_type=_f32)
        pd_ref[...] = 2.0 * dot - xx - jnp.transpose(xx)

        def body(j, acc):
            nv = pd_ref[...]
            rmax = jnp.max(nv, axis=1, keepdims=True)
            cand = jnp.where(nv == rmax, iota, N)
            am = jnp.min(cand, axis=1, keepdims=True)
            pd_ref[...] = jnp.where(iota == am, -jnp.inf, nv)
            return jnp.where(iok == j, am, acc)

        idxb = lax.fori_loop(0, K, body, jnp.zeros((N, K), jnp.int32))
        idx_ref[pl.ds(bb * N, N), :] = idxb + bb * N


def _first_body(x_ref, idx_ref, pd_ref):
    _topk_writeidx(x_ref[...], idx_ref, pd_ref)


def _mid_body(s1_ref, s2_ref, mx_ref, mn_ref, g_ref, b_ref,
              x_ref, idx_ref, pd_ref):
    xv = _epilogue(s1_ref[...], s2_ref[...], mx_ref[...], mn_ref[...],
                   g_ref[...], b_ref[...])
    x_ref[...] = xv
    _topk_writeidx(xv, idx_ref, pd_ref)


def _epi_body(s1_ref, s2_ref, mx_ref, mn_ref, g_ref, b_ref, x_ref):
    x_ref[...] = _epilogue(s1_ref[...], s2_ref[...], mx_ref[...], mn_ref[...],
                           g_ref[...], b_ref[...])


def _edge_body(gx_ref, x_ref, w_ref, s1_ref, s2_ref, mx_ref, mn_ref):
    k = pl.program_id(1)
    xv = x_ref[...]
    f = jnp.concatenate([gx_ref[...] - xv, xv], axis=1).astype(_bf16)
    y = jnp.dot(f, w_ref[...], preferred_element_type=_f32)
    y2 = y * y

    @pl.when(k == 0)
    def _():
        s1_ref[...] = y
        s2_ref[...] = y2
        mx_ref[...] = y
        mn_ref[...] = y

    @pl.when(k != 0)
    def _():
        s1_ref[...] += y
        s2_ref[...] += y2
        mx_ref[...] = jnp.maximum(mx_ref[...], y)
        mn_ref[...] = jnp.minimum(mn_ref[...], y)


def _final_body(x1_ref, x2_ref, x3_ref, x4_ref,
                w5a_ref, w5b_ref, w5c_ref, w5d_ref,
                g5_ref, b5_ref, gvec_ref, y_ref):
    for bb in range(B):
        sl = pl.ds(bb * N, N)
        yb = (jnp.dot(x1_ref[sl, :].astype(_bf16), w5a_ref[...],
                      preferred_element_type=_f32)
              + jnp.dot(x2_ref[sl, :].astype(_bf16), w5b_ref[...],
                        preferred_element_type=_f32)
              + jnp.dot(x3_ref[sl, :].astype(_bf16), w5c_ref[...],
                        preferred_element_type=_f32)
              + jnp.dot(x4_ref[sl, :].astype(_bf16), w5d_ref[...],
                        preferred_element_type=_f32))
        y_ref[sl, :] = yb
    ysum = jnp.zeros((1, 1024), _f32)
    for bb in range(B):
        ysum = ysum + jnp.sum(y_ref[pl.ds(bb * N, N), :], axis=0, keepdims=True)
    m = ysum * (1.0 / BN)
    vsum = jnp.zeros((1, 1024), _f32)
    for bb in range(B):
        d = y_ref[pl.ds(bb * N, N), :] - m
        vsum = vsum + jnp.sum(d * d, axis=0, keepdims=True)
    v = vsum * (1.0 / BN)
    g5 = g5_ref[...]
    b5 = b5_ref[...]
    inv = g5 / jnp.sqrt(v + 1e-5)
    for bb in range(B):
        blk = y_ref[pl.ds(bb * N, N), :]
        sel = jnp.where(g5 >= 0,
                        jnp.max(blk, axis=0, keepdims=True),
                        jnp.min(blk, axis=0, keepdims=True))
        p1 = _lrelu((sel - m) * inv + b5)
        act = _lrelu((blk - m) * inv + b5)
        p2 = jnp.sum(act, axis=0, keepdims=True) * (1.0 / N)
        gvec_ref[pl.ds(bb, 1), pl.ds(0, 1024)] = p1
        gvec_ref[pl.ds(bb, 1), pl.ds(1024, 1024)] = p2


def _tc_first(x0p):
    return pl.pallas_call(
        _first_body,
        out_shape=jax.ShapeDtypeStruct((BN, K), jnp.int32),
        scratch_shapes=[pltpu.VMEM((N, N), _f32)],
    )(x0p)


def _tc_mid(S1, S2, MX, MN, g, b):
    return pl.pallas_call(
        _mid_body,
        out_shape=(
            jax.ShapeDtypeStruct((BN, CP), _f32),
            jax.ShapeDtypeStruct((BN, K), jnp.int32),
        ),
        scratch_shapes=[pltpu.VMEM((N, N), _f32)],
    )(S1, S2, MX, MN, g, b)


def _tc_epi(S1, S2, MX, MN, g, b, opad):
    return pl.pallas_call(
        _epi_body,
        out_shape=jax.ShapeDtypeStruct((BN, opad), _f32),
    )(S1, S2, MX, MN, g, b)


def _tc_edge(gx, x, w16, opad):
    return pl.pallas_call(
        _edge_body,
        grid=(NPB, K),
        in_specs=[
            pl.BlockSpec((PB, CP), lambda i, k: (k * NPB + i, 0)),
            pl.BlockSpec((PB, CP), lambda i, k: (i, 0)),
            pl.BlockSpec((2 * CP, opad), lambda i, k: (0, 0)),
        ],
        out_specs=[
            pl.BlockSpec((PB, opad), lambda i, k: (i, 0)),
            pl.BlockSpec((PB, opad), lambda i, k: (i, 0)),
            pl.BlockSpec((PB, opad), lambda i, k: (i, 0)),
            pl.BlockSpec((PB, opad), lambda i, k: (i, 0)),
        ],
        out_shape=(
            jax.ShapeDtypeStruct((BN, opad), _f32),
            jax.ShapeDtypeStruct((BN, opad), _f32),
            jax.ShapeDtypeStruct((BN, opad), _f32),
            jax.ShapeDtypeStruct((BN, opad), _f32),
        ),
    )(gx, x, w16)


def _tc_final(x1, x2, x3, x4, w5a, w5b, w5c, w5d, g5, b5):
    return pl.pallas_call(
        _final_body,
        out_shape=jax.ShapeDtypeStruct((B, 2048), _f32),
        scratch_shapes=[pltpu.VMEM((BN, 1024), _f32)],
    )(x1, x2, x3, x4, w5a, w5b, w5c, w5d, g5, b5)


_NW = 32
_RW = BN * K // _NW
_GR = 128
_NG = _RW // _GR


def _sc_gather_body(x_hbm, idx_hbm, gx_hbm, idx_v, rows0_v, rows1_v,
                    sem0, sem1):
    wid = lax.axis_index("s") * 2 + lax.axis_index("c")
    base = wid * _RW
    pltpu.sync_copy(idx_hbm.at[pl.ds(base, _RW)], idx_v)

    def g_body(h, _):
        off0 = (2 * h) * _GR
        off1 = off0 + _GR
        cp0 = pltpu.async_copy(x_hbm.at[idx_v.at[pl.ds(off0, _GR)]],
                               rows0_v, sem0)
        cp1 = pltpu.async_copy(x_hbm.at[idx_v.at[pl.ds(off1, _GR)]],
                               rows1_v, sem1)
        cp0.wait()
        pltpu.sync_copy(rows0_v, gx_hbm.at[pl.ds(base + off0, _GR)])
        cp1.wait()
        pltpu.sync_copy(rows1_v, gx_hbm.at[pl.ds(base + off1, _GR)])
        return 0

    lax.fori_loop(0, _NG // 2, g_body, 0)


@functools.lru_cache(maxsize=None)
def _make_sc_gather():
    mesh = plsc.VectorSubcoreMesh(core_axis_name="c", subcore_axis_name="s")
    return functools.partial(
        pl.kernel,
        out_type=jax.ShapeDtypeStruct((BN * K, CP), _f32),
        mesh=mesh,
        scratch_types=[
            pltpu.VMEM((_RW,), jnp.int32),
            pltpu.VMEM((_GR, CP), _f32),
            pltpu.VMEM((_GR, CP), _f32),
            pltpu.SemaphoreType.DMA,
            pltpu.SemaphoreType.DMA,
        ],
    )(_sc_gather_body)


def _sc_gather(x, idx_km):
    return _make_sc_gather()(x, idx_km)


def _prep_w16(W, opad):
    O, twoC = W.shape
    C = twoC // 2
    w = jnp.zeros((2 * CP, opad), _f32)
    w = w.at[:C, :O].set(W[:, :C].T)
    w = w.at[CP:CP + C, :O].set(W[:, C:].T)
    return w.astype(_bf16)


def _pad_vec(g, opad):
    return jnp.zeros((1, opad), _f32).at[0, : g.shape[0]].set(g)


def _kmajor(idx):
    return jnp.transpose(idx).reshape(-1)


def kernel(xyz, W1, g1, b1, W2, g2, b2, W3, g3, b3, W4, g4, b4, W5, g5, b5):
    x0p = jnp.pad(xyz.reshape(BN, 3), ((0, 0), (0, CP - 3)))

    idx1 = _tc_first(x0p)
    gx = _sc_gather(x0p, _kmajor(idx1))
    S1, S2, MX, MN = _tc_edge(gx, x0p, _prep_w16(W1, 128), 128)

    x1, idx2 = _tc_mid(S1, S2, MX, MN, _pad_vec(g1, 128), _pad_vec(b1, 128))
    gx = _sc_gather(x1, _kmajor(idx2))
    S1, S2, MX, MN = _tc_edge(gx, x1, _prep_w16(W2, 128), 128)

    x2, idx3 = _tc_mid(S1, S2, MX, MN, _pad_vec(g2, 128), _pad_vec(b2, 128))
    gx = _sc_gather(x2, _kmajor(idx3))
    S1, S2, MX, MN = _tc_edge(gx, x2, _prep_w16(W3, 128), 128)

    x3, idx4 = _tc_mid(S1, S2, MX, MN, _pad_vec(g3, 128), _pad_vec(b3, 128))
    gx = _sc_gather(x3, _kmajor(idx4))
    S1, S2, MX, MN = _tc_edge(gx, x3, _prep_w16(W4, 384), 384)

    x4 = _tc_epi(S1, S2, MX, MN, _pad_vec(g4, 384), _pad_vec(b4, 384), 384)

    w5a = jnp.zeros((128, 1024), _f32).at[:64, :].set(W5[:, :64].T).astype(_bf16)
    w5b = jnp.zeros((128, 1024), _f32).at[:64, :].set(W5[:, 64:128].T).astype(_bf16)
    w5c = jnp.asarray(W5[:, 128:256].T, _f32).astype(_bf16)
    w5d = jnp.zeros((384, 1024), _f32).at[:257, :].set(W5[:, 256:513].T).astype(_bf16)

    gvec = _tc_final(x1, x2, x3, x4, w5a, w5b, w5c, w5d,
                     _pad_vec(g5, 1024), _pad_vec(b5, 1024))

    xf = jnp.concatenate([x1[:, :64], x2[:, :64], x3, x4[:, :257]], axis=1)
    xft = jnp.transpose(xf.reshape(B, N, 513), (0, 2, 1))
    x0t = jnp.transpose(xyz, (0, 2, 1))
    gr = jnp.broadcast_to(gvec[:, :, None], (B, 2048, N))
    return jnp.concatenate([gr, x0t, xft], axis=1)
